# Initial kernel scaffold; baseline (speedup 1.0000x reference)
import jax
import jax.numpy as jnp
from jax.experimental import pallas as pl


def kernel(depth, label):
    raise NotImplementedError("write your pallas kernel here")


# speedup vs baseline: 166.1781x; 166.1781x over previous
import math

import jax
import jax.numpy as jnp
from jax.experimental import pallas as pl

_NUM_CLASSES = 20
_KNN = 5
_CUTOFF = 1.0

_V = [math.exp(-(i - 2) ** 2 / 2.0) for i in range(5)]
_V = [x / sum(_V) for x in _V]


def _body(dp_ref, lp_ref, out_ref):
    H, W = out_ref.shape[1], out_ref.shape[2]
    dp = dp_ref[0]
    lp = lp_ref[0]

    JH, JW = H + 4, W + 4
    base = dp[2:2 + JH, 2:2 + JW]
    rows = jax.lax.broadcasted_iota(jnp.int32, (JH, JW), 0)
    cols = jax.lax.broadcasted_iota(jnp.int32, (JH, JW), 1)
    inimg = (rows >= 2) & (rows < JH - 2) & (cols >= 2) & (cols < JW - 2)

    slots_d = []
    slots_l = []

    for k in range(25):
        dh, dw = k // 5 - 2, k % 5 - 2
        lab = lp[2 + dh:2 + dh + H, 2 + dw:2 + dw + W]
        if dh == 0 and dw == 0:
            dist = jnp.zeros((H, W), jnp.float32)
        else:
            nb = dp[2 + dh:2 + dh + JH, 2 + dw:2 + dw + JW]
            jp = jnp.where(inimg, jnp.abs(nb - base), 0.0)
            rb = jp[:, 0:W]
            rg = _V[0] * rb
            for j in range(1, 5):
                t = jp[:, j:j + W]
                rb = rb + t
                rg = rg + _V[j] * t
            bx = rb[0:H]
            gs = _V[0] * rg[0:H]
            for i in range(1, 5):
                bx = bx + rb[i:i + H]
                gs = gs + _V[i] * rg[i:i + H]
            dist = bx - gs

        if len(slots_d) < _KNN:
            slots_d.append(dist)
            slots_l.append(lab)
        else:
            cd, cl = dist, lab
            for i in range(_KNN):
                c = cd < slots_d[i]
                nd = jnp.minimum(slots_d[i], cd)
                cd = jnp.maximum(slots_d[i], cd)
                nl = jnp.where(c, cl, slots_l[i])
                cl = jnp.where(c, slots_l[i], cl)
                slots_d[i] = nd
                slots_l[i] = nl

    ls = [jnp.where(slots_d[i] > _CUTOFF, float(_NUM_CLASSES), slots_l[i])
          for i in range(_KNN)]

    ones = jnp.ones_like(ls[0])
    zeros = jnp.zeros_like(ls[0])
    cnt = [ones, ones, ones, ones, ones]
    for i in range(_KNN):
        for j in range(i + 1, _KNN):
            e = jnp.where(ls[i] == ls[j], 1.0, 0.0)
            cnt[i] = cnt[i] + e
            cnt[j] = cnt[j] + e
    key = zeros
    for i in range(_KNN):
        ki = jnp.where(ls[i] == float(_NUM_CLASSES), 0.0,
                       cnt[i] * 32.0 + (31.0 - ls[i]))
        key = jnp.maximum(key, ki)
    m = key - 32.0 * jnp.floor(key * (1.0 / 32.0))
    best = jnp.where(key > 0.0, 31.0 - m, 0.0)
    out_ref[0] = best.astype(jnp.int32)


def kernel(depth, label):
    B, C, H, W = depth.shape
    d = depth[:, 0]
    dp = jnp.pad(d, ((0, 0), (4, 4), (4, 4)))
    lp = jnp.pad(label.astype(jnp.float32), ((0, 0), (2, 2), (2, 2)))
    return pl.pallas_call(
        _body,
        grid=(B,),
        in_specs=[
            pl.BlockSpec((1, H + 8, W + 8), lambda b: (b, 0, 0)),
            pl.BlockSpec((1, H + 4, W + 4), lambda b: (b, 0, 0)),
        ],
        out_specs=pl.BlockSpec((1, H, W), lambda b: (b, 0, 0)),
        out_shape=jax.ShapeDtypeStruct((B, H, W), jnp.int32),
    )(dp, lp)


# speedup vs baseline: 202.6906x; 1.2197x over previous
import math

import jax
import jax.numpy as jnp
from jax.experimental import pallas as pl

_NUM_CLASSES = 20
_CUTOFF = 1.0

_V = [math.exp(-(i - 2) ** 2 / 2.0) for i in range(5)]
_V = [x / sum(_V) for x in _V]


def _dot(m, x):
    return jax.lax.dot_general(
        m, x, (((1,), (0,)), ((), ())),
        precision=jax.lax.Precision.HIGHEST,
        preferred_element_type=jnp.float32)


def _body(dp_ref, lp_ref, out_ref):
    H, W = out_ref.shape[1], out_ref.shape[2]
    dp = dp_ref[0]
    lp = lp_ref[0]

    JH, JW = H + 4, W + 4
    base = dp[2:2 + JH, 2:2 + JW]

    cols = jax.lax.broadcasted_iota(jnp.int32, (1, JW), 1)
    colmask = jnp.where((cols >= 2) & (cols < JW - 2), 1.0, 0.0)

    hh = jax.lax.broadcasted_iota(jnp.int32, (H, JH), 0)
    rr = jax.lax.broadcasted_iota(jnp.int32, (H, JH), 1)
    dlt = rr - hh
    rowok = (rr >= 2) & (rr < JH - 2)
    mb = jnp.where((dlt >= 0) & (dlt <= 4) & rowok, 1.0, 0.0)
    mg = jnp.zeros((H, JH), jnp.float32)
    for i in range(5):
        mg = jnp.where((dlt == i) & rowok, _V[i], mg)

    slots_d = []
    slots_l = []

    for k in range(25):
        if k == 12:
            continue
        dh, dw = k // 5 - 2, k % 5 - 2
        lab = lp[2 + dh:2 + dh + H, 2 + dw:2 + dw + W]
        nb = dp[2 + dh:2 + dh + JH, 2 + dw:2 + dw + JW]
        jp = jnp.abs(nb - base) * colmask
        t0 = jp[:, 0:W]
        t1 = jp[:, 1:1 + W]
        t2 = jp[:, 2:2 + W]
        t3 = jp[:, 3:3 + W]
        t4 = jp[:, 4:4 + W]
        s04 = t0 + t4
        s13 = t1 + t3
        rb = s04 + s13 + t2
        rg = _V[0] * s04 + _V[1] * s13 + _V[2] * t2
        dist = _dot(mb, rb) - _dot(mg, rg)

        if len(slots_d) < 4:
            slots_d.append(dist)
            slots_l.append(lab)
        else:
            cd, cl = dist, lab
            for i in range(4):
                c = cd < slots_d[i]
                nd = jnp.minimum(slots_d[i], cd)
                cd = jnp.maximum(slots_d[i], cd)
                nl = jnp.where(c, cl, slots_l[i])
                cl = jnp.where(c, slots_l[i], cl)
                slots_d[i] = nd
                slots_l[i] = nl

    ls = [lp[2:2 + H, 2:2 + W]]
    ls += [jnp.where(slots_d[i] > _CUTOFF, float(_NUM_CLASSES), slots_l[i])
           for i in range(4)]

    ones = jnp.ones_like(ls[0])
    cnt = [ones, ones, ones, ones, ones]
    for i in range(5):
        for j in range(i + 1, 5):
            e = jnp.where(ls[i] == ls[j], 1.0, 0.0)
            cnt[i] = cnt[i] + e
            cnt[j] = cnt[j] + e
    key = jnp.zeros_like(ls[0])
    for i in range(5):
        ki = jnp.where(ls[i] == float(_NUM_CLASSES), 0.0,
                       cnt[i] * 32.0 + (31.0 - ls[i]))
        key = jnp.maximum(key, ki)
    m = key - 32.0 * jnp.floor(key * (1.0 / 32.0))
    best = jnp.where(key > 0.0, 31.0 - m, 0.0)
    out_ref[0] = best.astype(jnp.int32)


def kernel(depth, label):
    B, C, H, W = depth.shape
    d = depth[:, 0]
    dp = jnp.pad(d, ((0, 0), (4, 4), (4, 4)))
    lp = jnp.pad(label.astype(jnp.float32), ((0, 0), (2, 2), (2, 2)))
    return pl.pallas_call(
        _body,
        grid=(B,),
        in_specs=[
            pl.BlockSpec((1, H + 8, W + 8), lambda b: (b, 0, 0)),
            pl.BlockSpec((1, H + 4, W + 4), lambda b: (b, 0, 0)),
        ],
        out_specs=pl.BlockSpec((1, H, W), lambda b: (b, 0, 0)),
        out_shape=jax.ShapeDtypeStruct((B, H, W), jnp.int32),
    )(dp, lp)


# speedup vs baseline: 208.0699x; 1.0265x over previous
import math

import jax
import jax.numpy as jnp
from jax.experimental import pallas as pl

_NUM_CLASSES = 20
_CUTOFF = 1.0

_V = [math.exp(-(i - 2) ** 2 / 2.0) for i in range(5)]
_V = [x / sum(_V) for x in _V]


def _dot(m, x):
    return jax.lax.dot_general(
        m, x, (((1,), (0,)), ((), ())),
        precision=jax.lax.Precision.HIGHEST,
        preferred_element_type=jnp.float32)


def _body(dp_ref, lp_ref, out_ref):
    H, W = out_ref.shape[1], out_ref.shape[2]
    dp = dp_ref[0]
    lp = lp_ref[0]

    JH, JW = H + 4, W + 4
    base = dp[2:2 + JH, 2:2 + JW]

    cols = jax.lax.broadcasted_iota(jnp.int32, (1, JW), 1)
    colmask = jnp.where((cols >= 2) & (cols < JW - 2), 1.0, 0.0)

    hh = jax.lax.broadcasted_iota(jnp.int32, (H, JH), 0)
    rr = jax.lax.broadcasted_iota(jnp.int32, (H, JH), 1)
    dlt = rr - hh
    rowok = (rr >= 2) & (rr < JH - 2)
    mb = jnp.where((dlt >= 0) & (dlt <= 4) & rowok, 1.0, 0.0)
    mg = jnp.zeros((H, JH), jnp.float32)
    for i in range(5):
        mg = jnp.where((dlt == i) & rowok, _V[i], mg)

    slots = []

    for k in range(25):
        if k == 12:
            continue
        dh, dw = k // 5 - 2, k % 5 - 2
        lab = lp[2 + dh:2 + dh + H, 2 + dw:2 + dw + W]
        nb = dp[2 + dh:2 + dh + JH, 2 + dw:2 + dw + JW]
        jp = jnp.abs(nb - base) * colmask
        t0 = jp[:, 0:W]
        t1 = jp[:, 1:1 + W]
        t2 = jp[:, 2:2 + W]
        t3 = jp[:, 3:3 + W]
        t4 = jp[:, 4:4 + W]
        s04 = t0 + t4
        s13 = t1 + t3
        rb = s04 + s13 + t2
        rg = _V[0] * s04 + _V[1] * s13 + _V[2] * t2
        dist = _dot(mb, rb) - _dot(mg, rg)

        key = (jax.lax.bitcast_convert_type(dist, jnp.int32)
               & jnp.int32(-32)) | lab
        if len(slots) < 4:
            slots.append(key)
        else:
            ck = key
            for i in range(4):
                nk = jnp.minimum(slots[i], ck)
                ck = jnp.maximum(slots[i], ck)
                slots[i] = nk

    cut = jnp.int32(0x3F800000)
    ls = [lp[2:2 + H, 2:2 + W]]
    for i in range(4):
        di = slots[i] & jnp.int32(-32)
        ls.append(jnp.where(di > cut, _NUM_CLASSES, slots[i] & 31))

    ones = jnp.ones_like(ls[0])
    cnt = [ones, ones, ones, ones, ones]
    for i in range(5):
        for j in range(i + 1, 5):
            e = jnp.where(ls[i] == ls[j], 1, 0)
            cnt[i] = cnt[i] + e
            cnt[j] = cnt[j] + e
    key = jnp.zeros_like(ls[0])
    for i in range(5):
        ki = jnp.where(ls[i] == _NUM_CLASSES, 0,
                       cnt[i] * 32 + (31 - ls[i]))
        key = jnp.maximum(key, ki)
    best = jnp.where(key > 0, 31 - (key & 31), 0)
    out_ref[0] = best


def kernel(depth, label):
    B, C, H, W = depth.shape
    d = depth[:, 0]
    dp = jnp.pad(d, ((0, 0), (4, 4), (4, 4)))
    lp = jnp.pad(label, ((0, 0), (2, 2), (2, 2)))
    return pl.pallas_call(
        _body,
        grid=(B,),
        in_specs=[
            pl.BlockSpec((1, H + 8, W + 8), lambda b: (b, 0, 0)),
            pl.BlockSpec((1, H + 4, W + 4), lambda b: (b, 0, 0)),
        ],
        out_specs=pl.BlockSpec((1, H, W), lambda b: (b, 0, 0)),
        out_shape=jax.ShapeDtypeStruct((B, H, W), jnp.int32),
    )(dp, lp)
